# baseline (device time: 66738 ns/iter reference)
import jax
import jax.numpy as jnp
from jax import lax
from jax.experimental import pallas as pl
from jax.experimental.pallas import tpu as pltpu

N_DEV = 16
F8 = jnp.float8_e4m3fn
N_BLOCKS = 8


def kernel(x, w_mat, scale_x, scale_w):
    m_total, k_shard = x.shape
    k_total, n_total = w_mat.shape
    m_blk = m_total // N_DEV
    n_blk = n_total // N_BLOCKS

    def body(x_ref, w_ref, sx_ref, sw_ref, out_ref,
             x8_ref, recv_ref, xg_ref, send_sems, recv_sems):
        c = pl.program_id(0)
        me = lax.axis_index("i")

        @pl.when(c == 0)
        def _a2a():
            x8_ref[...] = x_ref[...].astype(F8)

            bsem = pltpu.get_barrier_semaphore()
            for d in range(N_DEV):
                pl.semaphore_signal(
                    bsem, inc=1,
                    device_id=(d,), device_id_type=pl.DeviceIdType.MESH,
                )
            pl.semaphore_wait(bsem, N_DEV)

            for off in range(1, N_DEV):
                d = (me + off) % N_DEV
                rdma = pltpu.make_async_remote_copy(
                    src_ref=x8_ref.at[pl.ds(d * m_blk, m_blk), :],
                    dst_ref=recv_ref.at[me],
                    send_sem=send_sems.at[d],
                    recv_sem=recv_sems.at[me],
                    device_id=(d,),
                    device_id_type=pl.DeviceIdType.MESH,
                )
                rdma.start()

            xg_ref[:, pl.ds(me * m_blk, m_blk)] = (
                x8_ref[pl.ds(me * m_blk, m_blk), :]
            )

            for off in range(1, N_DEV):
                jj = (me + off) % N_DEV
                recv = pltpu.make_async_remote_copy(
                    src_ref=x8_ref.at[pl.ds(0, m_blk), :],
                    dst_ref=recv_ref.at[jj],
                    send_sem=send_sems.at[jj],
                    recv_sem=recv_sems.at[jj],
                    device_id=(me,),
                    device_id_type=pl.DeviceIdType.MESH,
                )
                recv.wait_recv()
                xg_ref[:, pl.ds(jj * m_blk, m_blk)] = recv_ref[jj]

        w8 = w_ref[...].astype(F8)
        acc = jnp.dot(xg_ref[...], w8, preferred_element_type=jnp.float32)
        s = sx_ref[0] * sw_ref[0]
        out_ref[...] = jnp.maximum(acc * s, 0.0)

        @pl.when(c == N_BLOCKS - 1)
        def _drain():
            for off in range(1, N_DEV):
                d = (me + off) % N_DEV
                sd = pltpu.make_async_remote_copy(
                    src_ref=x8_ref.at[pl.ds(d * m_blk, m_blk), :],
                    dst_ref=recv_ref.at[me],
                    send_sem=send_sems.at[d],
                    recv_sem=recv_sems.at[me],
                    device_id=(d,),
                    device_id_type=pl.DeviceIdType.MESH,
                )
                sd.wait_send()

    return pl.pallas_call(
        body,
        grid=(N_BLOCKS,),
        out_shape=jax.ShapeDtypeStruct((m_blk, n_total), jnp.float32),
        in_specs=[
            pl.BlockSpec((m_total, k_shard), lambda c: (0, 0)),
            pl.BlockSpec((k_total, n_blk), lambda c: (0, c)),
            pl.BlockSpec(memory_space=pltpu.SMEM),
            pl.BlockSpec(memory_space=pltpu.SMEM),
        ],
        out_specs=pl.BlockSpec((m_blk, n_blk), lambda c: (0, c)),
        scratch_shapes=[
            pltpu.VMEM((m_total, k_shard), F8),
            pltpu.VMEM((N_DEV, m_blk, k_shard), F8),
            pltpu.VMEM((m_blk, k_total), F8),
            pltpu.SemaphoreType.DMA((N_DEV,)),
            pltpu.SemaphoreType.DMA((N_DEV,)),
        ],
        compiler_params=pltpu.CompilerParams(
            dimension_semantics=("arbitrary",),
            collective_id=0,
            has_side_effects=True,
            vmem_limit_bytes=64 * 1024 * 1024,
        ),
    )(x, w_mat, scale_x, scale_w)
